# baseline (speedup 1.0000x reference)
import functools
import math

import jax
import jax.numpy as jnp
from jax import lax
from jax.experimental import pallas as pl
from jax.experimental.pallas import tpu as pltpu
from jax.experimental.pallas import tpu_sc as plsc

_D = 64
_SCALE = 8.0
_LANES = 16


@functools.cache
def _make_sc_gather(S0, S1, V, D, chunk):
    B = S0 * S1
    NC, NS = 2, 16
    NW = NC * NS
    b_per_w = B // NW
    assert b_per_w * NW == B and b_per_w % chunk == 0
    assert chunk % S1 == 0 and chunk % _LANES == 0
    rows_per_chunk = chunk // S1
    n_chunks = b_per_w // chunk
    mesh = plsc.VectorSubcoreMesh(core_axis_name="c", subcore_axis_name="s")

    @functools.partial(
        pl.kernel,
        out_type=jax.ShapeDtypeStruct((S0, S1, D), jnp.float32),
        mesh=mesh,
        scratch_types=[
            pltpu.VMEM((chunk,), jnp.int32),
            pltpu.VMEM((chunk,), jnp.int32),
            pltpu.VMEM((chunk, 2 * D), jnp.float32),
            pltpu.VMEM((chunk, D), jnp.float32),
            pltpu.SemaphoreType.DMA,
            pltpu.SemaphoreType.DMA,
        ],
        compiler_params=pltpu.CompilerParams(needs_layout_passes=False),
    )
    def sc_gather(x_hbm, table_hbm, out_hbm, idx_v, pidx_v, big_v, comp_v,
                  sg, ss):
        wid = lax.axis_index("s") * NC + lax.axis_index("c")
        base = wid * b_per_w
        lane = lax.iota(jnp.int32, _LANES)

        def chunk_body(g, _):
            off = base + g * chunk
            pltpu.sync_copy(x_hbm.at[pl.ds(off, chunk)], idx_v)

            def pidx_body(i, _):
                sl = pl.ds(i * _LANES, _LANES)
                pidx_v[sl] = lax.shift_right_logical(idx_v[sl], 1)
                return ()

            lax.fori_loop(0, chunk // _LANES, pidx_body, ())

            pltpu.async_copy(table_hbm.at[pidx_v], big_v, sg).wait()

            def sel_body(i, _):
                sl = pl.ds(i * _LANES, _LANES)
                v = idx_v[sl]
                rvec = i * _LANES + lane
                half = (v & 1) * D
                for c in range(D):
                    vals = plsc.load_gather(big_v, [rvec, half + c])
                    plsc.store_scatter(comp_v, [rvec, lane * 0 + c],
                                       vals * _SCALE)
                return ()

            lax.fori_loop(0, chunk // _LANES, sel_body, ())

            xr0 = off // S1
            for k in range(rows_per_chunk):
                pltpu.async_copy(
                    comp_v.at[pl.ds(k * S1, S1)], out_hbm.at[xr0 + k], ss)
            for k in range(rows_per_chunk):
                pltpu.make_async_copy(
                    comp_v.at[pl.ds(k * S1, S1)], out_hbm.at[xr0 + k], ss).wait()
            return ()

        lax.fori_loop(0, n_chunks, chunk_body, ())

    return sc_gather


def kernel(x, table):
    S0, S1 = x.shape
    V, D = table.shape
    flat = x.reshape(S0 * S1).astype(jnp.int32)
    table2 = table.reshape(V // 2, 2 * D)
    return _make_sc_gather(S0, S1, V, D, 400)(flat, table2)


# speedup vs baseline: 2.4240x; 2.4240x over previous
import functools
import math

import jax
import jax.numpy as jnp
from jax import lax
from jax.experimental import pallas as pl
from jax.experimental.pallas import tpu as pltpu
from jax.experimental.pallas import tpu_sc as plsc

_D = 64
_SCALE = 8.0
_LANES = 16


@functools.cache
def _make_sc_gather(S0, S1, V, D, chunk):
    B = S0 * S1
    NC, NS = 2, 16
    NW = NC * NS
    b_per_w = B // NW
    assert b_per_w * NW == B and b_per_w % chunk == 0 and chunk % 2 == 0
    n_chunks = b_per_w // chunk
    hchunk = chunk // 2
    mesh = plsc.VectorSubcoreMesh(core_axis_name="c", subcore_axis_name="s")

    @functools.partial(
        pl.kernel,
        out_type=jax.ShapeDtypeStruct((B // 2, 2 * D), jnp.float32),
        mesh=mesh,
        scratch_types=[
            pltpu.VMEM((chunk,), jnp.int32),
            pltpu.VMEM((chunk,), jnp.int32),
            pltpu.VMEM((chunk, D), jnp.float32),
            pltpu.VMEM((chunk, D), jnp.float32),
            pltpu.VMEM((hchunk, 2 * D), jnp.float32),
            pltpu.VMEM((hchunk, 2 * D), jnp.float32),
            pltpu.SemaphoreType.DMA,
            pltpu.SemaphoreType.DMA,
            pltpu.SemaphoreType.DMA,
            pltpu.SemaphoreType.DMA,
        ],
        compiler_params=pltpu.CompilerParams(use_tc_tiling_on_sc=False),
    )
    def sc_gather(x_hbm, table_hbm, out_hbm, idx0, idx1, big0, big1,
                  comp0, comp1, sg0, sg1, ss0, ss1):
        wid = lax.axis_index("s") * NC + lax.axis_index("c")
        base = wid * b_per_w
        slots = ((idx0, big0, comp0, sg0, ss0), (idx1, big1, comp1, sg1, ss1))

        def start_gather(g, slot):
            idx, big, _, sg, _ = slot
            pltpu.sync_copy(x_hbm.at[pl.ds(base + g * chunk, chunk)], idx)
            pltpu.async_copy(table_hbm.at[idx], big, sg)

        def wait_gather(slot):
            idx, big, _, sg, _ = slot
            pltpu.make_async_copy(table_hbm.at[idx], big, sg).wait()

        def scale_pack(slot):
            _, big, comp, _, _ = slot

            def pair_body(p, _):
                for h in range(2):
                    for j in range(D // _LANES):
                        src = pl.ds(j * _LANES, _LANES)
                        dst = pl.ds(h * D + j * _LANES, _LANES)
                        comp[p, dst] = big[2 * p + h, src] * _SCALE
                return ()

            lax.fori_loop(0, hchunk, pair_body, (), unroll=4)

        def start_scatter(g, slot):
            _, _, comp, _, ss = slot
            ph0 = (base + g * chunk) // 2
            pltpu.async_copy(comp, out_hbm.at[pl.ds(ph0, hchunk)], ss)

        def wait_scatter(g, slot):
            _, _, comp, _, ss = slot
            ph0 = (base + g * chunk) // 2
            pltpu.make_async_copy(
                comp, out_hbm.at[pl.ds(ph0, hchunk)], ss).wait()

        start_gather(0, slots[0])

        def pair(p, _):
            for b in range(2):
                g = p * 2 + b
                nslot = slots[1 - b]

                @pl.when(g + 1 < n_chunks)
                def _():
                    start_gather(g + 1, nslot)

                wait_gather(slots[b])

                @pl.when(g >= 2)
                def _():
                    wait_scatter(g - 2, slots[b])

                scale_pack(slots[b])
                start_scatter(g, slots[b])
            return ()

        lax.fori_loop(0, n_chunks // 2, pair, ())
        wait_scatter(n_chunks - 2, slots[0])
        wait_scatter(n_chunks - 1, slots[1])

    return sc_gather


def kernel(x, table):
    S0, S1 = x.shape
    V, D = table.shape
    flat = x.reshape(S0 * S1).astype(jnp.int32)
    out2 = _make_sc_gather(S0, S1, V, D, 400)(flat, table)
    return out2.reshape(S0, S1, D)
